# Initial kernel scaffold; baseline (speedup 1.0000x reference)
import jax
import jax.numpy as jnp
from jax.experimental import pallas as pl


def kernel(x, table, W, b):
    raise NotImplementedError("write your pallas kernel here")


# speedup vs baseline: 15.6907x; 15.6907x over previous
import functools

import jax
import jax.numpy as jnp
from jax import lax
from jax.experimental import pallas as pl
from jax.experimental.pallas import tpu as pltpu
from jax.experimental.pallas import tpu_sc as plsc

B = 16384
FW = 20
ED = 32
NW = 32
ROWS_W = B // NW
R = 64
NCHUNK = ROWS_W // R
GPC = R * FW // 128
XROWS_W = ROWS_W * FW // 128


def _drain(tab_hbm, dbuf, sem):
    pltpu.make_async_copy(tab_hbm.at[pl.ds(0, R * FW)], dbuf, sem).wait()


def _sc_body(x_hbm, tab_hbm, w_hbm, b_hbm, out_hbm,
             idx0, idx1, d0, d1, w_v, b_v, out_v, sem0, sem1):
    cid = lax.axis_index("c")
    sid = lax.axis_index("s")
    wid = sid * 2 + cid
    base = wid * ROWS_W
    xrow0 = wid * XROWS_W

    pltpu.sync_copy(w_hbm, w_v)
    pltpu.sync_copy(b_hbm, b_v.at[pl.ds(0, 1)])
    bs = b_v[pl.ds(0, 16)][0]

    def stage_fire(cix, idxbuf, dbuf, sem):
        pltpu.sync_copy(x_hbm.at[wid * NCHUNK + cix], idxbuf)
        for j in range(GPC):
            pltpu.async_copy(tab_hbm.at[idxbuf.at[j]],
                             dbuf.at[pl.ds(j * 128, 128)], sem)

    lane = lax.broadcasted_iota(jnp.int32, (16,), 0)
    perms = [lane ^ (1 << k) for k in range(4)]
    bits = [(lane >> k) & 1 for k in range(4)]

    def _xsh(v, k):
        return v.at[perms[k]].get(mode="promise_in_bounds")

    def compute(cix, dbuf):
        def blk_body(blk, carry):
            accs = [jnp.zeros((16,), jnp.float32) for _ in range(16)]
            rbase = blk * (16 * FW)
            for k in range(2 * FW):
                wk = w_v[pl.ds(k * 16, 16)]
                rh = k // 2
                off = (k % 2) * 16
                for j in range(16):
                    d = dbuf[rbase + j * FW + rh, pl.ds(off, 16)]
                    accs[j] = accs[j] + d * wk
            vs = accs
            for k in range(4):
                nxt = []
                for p in range(len(vs) // 2):
                    a, b = vs[2 * p], vs[2 * p + 1]
                    nxt.append(jnp.where(bits[k] == 0,
                                         a + _xsh(a, k), b + _xsh(b, k)))
                vs = nxt
            out_v[pl.ds(blk * 16, 16)] = jnp.exp(vs[0] + bs)
            return carry

        lax.fori_loop(0, R // 16, blk_body, 0)
        pltpu.sync_copy(out_v, out_hbm.at[pl.ds(base + cix * R, R)])

    stage_fire(0, idx0, d0, sem0)

    def loop_body(t, carry):
        c0 = 2 * t
        stage_fire(c0 + 1, idx1, d1, sem1)
        _drain(tab_hbm, d0, sem0)
        compute(c0, d0)

        @pl.when(t < NCHUNK // 2 - 1)
        def _():
            stage_fire(c0 + 2, idx0, d0, sem0)

        _drain(tab_hbm, d1, sem1)
        compute(c0 + 1, d1)
        return carry

    lax.fori_loop(0, NCHUNK // 2, loop_body, 0)


_sc_call = functools.partial(
    pl.kernel,
    out_type=jax.ShapeDtypeStruct((B,), jnp.float32),
    mesh=plsc.VectorSubcoreMesh(core_axis_name="c", subcore_axis_name="s"),
    compiler_params=pltpu.CompilerParams(use_tc_tiling_on_sc=False),
    scratch_types=[
        pltpu.VMEM((GPC, 128), jnp.int32),
        pltpu.VMEM((GPC, 128), jnp.int32),
        pltpu.VMEM((R * FW, ED), jnp.float32),
        pltpu.VMEM((R * FW, ED), jnp.float32),
        pltpu.VMEM((ED * FW,), jnp.float32),
        pltpu.VMEM((16,), jnp.float32),
        pltpu.VMEM((R,), jnp.float32),
        pltpu.SemaphoreType.DMA,
        pltpu.SemaphoreType.DMA,
    ],
)(_sc_body)


@jax.jit
def kernel(x, table, W, b):
    x2 = x.astype(jnp.int32).reshape(NW * NCHUNK, GPC, 128)
    return _sc_call(x2, table, W.reshape(ED * FW), b)
